# Initial kernel scaffold; baseline (speedup 1.0000x reference)
import jax
import jax.numpy as jnp
from jax.experimental import pallas as pl


def kernel(x, edge_index, edge_weight, W1, b1, W2, b2):
    raise NotImplementedError("write your pallas kernel here")


# speedup vs baseline: 16.4512x; 16.4512x over previous
import functools

import jax
import jax.numpy as jnp
from jax import lax
from jax.experimental import pallas as pl
from jax.experimental.pallas import tpu as pltpu
from jax.experimental.pallas import tpu_sc as plsc

NC = 2
NS = 16
L = 16
NW = NC * NS


def _pick_chunk(ew_per_worker):
    for ch in (128, 112, 96, 80, 64, 48, 32, 16):
        if ew_per_worker % ch == 0:
            return ch
    return None


def _build_sc_scalar_agg(e, n, np_, gather):
    ew_per_worker = e // NW
    ch = _pick_chunk(ew_per_worker)
    nchunk = ew_per_worker // ch
    mesh = plsc.VectorSubcoreMesh(core_axis_name="c", subcore_axis_name="s")

    scratch = [
        pltpu.VMEM((ch,), jnp.int32),
        pltpu.VMEM((ch,), jnp.float32),
        pltpu.VMEM((np_,), jnp.float32),
    ]
    if gather:
        scratch += [
            pltpu.VMEM((ch,), jnp.int32),
            pltpu.VMEM((n,), jnp.float32),
        ]

    def body(*refs):
        if gather:
            (src_hbm, dst_hbm, ew_hbm, tab_hbm, out_hbm,
             didx, evals, acc, sidx, tab) = refs
        else:
            (dst_hbm, ew_hbm, out_hbm, didx, evals, acc) = refs
        cid = lax.axis_index("c")
        sid = lax.axis_index("s")
        wid = sid * NC + cid
        base = wid * ew_per_worker

        def zero(i, carry):
            acc[pl.ds(i * L, L)] = jnp.zeros((L,), jnp.float32)
            return carry
        lax.fori_loop(0, np_ // L, zero, 0)

        if gather:
            pltpu.sync_copy(tab_hbm, tab)

        def chunk(ci, carry):
            off = base + ci * ch
            pltpu.sync_copy(dst_hbm.at[pl.ds(off, ch)], didx)
            pltpu.sync_copy(ew_hbm.at[pl.ds(off, ch)], evals)
            if gather:
                pltpu.sync_copy(src_hbm.at[pl.ds(off, ch)], sidx)
            for g in range(ch // L):
                dv = didx[pl.ds(g * L, L)]
                ev = evals[pl.ds(g * L, L)]
                if gather:
                    sv = sidx[pl.ds(g * L, L)]
                    ev = ev * plsc.load_gather(tab, [sv])
                plsc.addupdate_scatter(acc, [dv], ev)
            return carry
        lax.fori_loop(0, nchunk, chunk, 0)

        pltpu.sync_copy(acc, out_hbm.at[wid])

    return pl.kernel(
        body,
        out_type=jax.ShapeDtypeStruct((NW, np_), jnp.float32),
        mesh=mesh,
        scratch_types=scratch,
        compiler_params=pltpu.CompilerParams(needs_layout_passes=False),
    )


def _build_sc_row_agg(e, n, np_, h):
    ew_per_worker = e // NW
    ch = _pick_chunk(ew_per_worker)
    nchunk = ew_per_worker // ch
    rows_per_tile = np_ // NS
    zch = rows_per_tile
    for z in (80, 64, 40, 32, 16, 8):
        if rows_per_tile % z == 0:
            zch = z
            break
    nz = rows_per_tile // zch
    mesh = plsc.VectorSubcoreMesh(core_axis_name="c", subcore_axis_name="s")

    scratch = [
        pltpu.VMEM((ch,), jnp.int32),
        pltpu.VMEM((ch,), jnp.int32),
        pltpu.VMEM((ch,), jnp.float32),
        pltpu.VMEM((ch, h), jnp.float32),
        pltpu.VMEM((zch, h), jnp.float32),
        pltpu.VMEM_SHARED((np_, h), jnp.float32),
        pltpu.SemaphoreType.DMA,
    ]

    def body(src_hbm, dst_hbm, ew_hbm, hp_hbm, out_hbm,
             sidx, didx, evals, rows, zb, spacc, sem):
        cid = lax.axis_index("c")
        sid = lax.axis_index("s")
        wid = sid * NC + cid
        base = wid * ew_per_worker

        def zrow(i, carry):
            zb[i, :] = jnp.zeros((h,), jnp.float32)
            return carry
        lax.fori_loop(0, zch, zrow, 0)

        def zcopy(k, carry):
            pltpu.sync_copy(zb, spacc.at[pl.ds(sid * rows_per_tile + k * zch, zch)])
            return carry
        lax.fori_loop(0, nz, zcopy, 0)
        plsc.subcore_barrier()

        def chunk(ci, carry):
            off = base + ci * ch
            pltpu.sync_copy(src_hbm.at[pl.ds(off, ch)], sidx)
            pltpu.sync_copy(dst_hbm.at[pl.ds(off, ch)], didx)
            pltpu.sync_copy(ew_hbm.at[pl.ds(off, ch)], evals)
            pltpu.async_copy(hp_hbm.at[sidx], rows, sem).wait()
            for g in range(ch // L):
                ev = evals[pl.ds(g * L, L)]
                for j in range(L):
                    i = g * L + j
                    rows[i, :] = rows[i, :] * ev[j]
            pltpu.sync_copy(rows, spacc.at[didx], add=True)
            return carry
        lax.fori_loop(0, nchunk, chunk, 0)
        plsc.subcore_barrier()

        pltpu.sync_copy(
            spacc.at[pl.ds(sid * rows_per_tile, rows_per_tile)],
            out_hbm.at[cid, pl.ds(sid * rows_per_tile, rows_per_tile)],
        )

    return pl.kernel(
        body,
        out_type=jax.ShapeDtypeStruct((NC, np_, h), jnp.float32),
        mesh=mesh,
        scratch_types=scratch,
        compiler_params=pltpu.CompilerParams(
            needs_layout_passes=False, use_tc_tiling_on_sc=False),
    )


def _tc_call(body, out_shapes):
    return pl.pallas_call(
        body,
        out_shape=out_shapes,
    )


def _tc_matmul(x, w):
    def body(x_ref, w_ref, o_ref):
        o_ref[...] = jnp.dot(x_ref[...], w_ref[...],
                             preferred_element_type=jnp.float32)
    n, _ = x.shape
    h = w.shape[1]
    return _tc_call(body, jax.ShapeDtypeStruct((n, h), jnp.float32))(x, w)


def _tc_norm(degp, hmat, n):
    def body(degp_ref, h_ref, dinv_ref, hp_ref):
        deg = jnp.sum(degp_ref[...], axis=0) + 1.0
        dinv = lax.rsqrt(deg)[:n]
        dinv_ref[...] = dinv[:, None]
        hp_ref[...] = h_ref[...] * dinv[:, None]
    hdim = hmat.shape[1]
    return _tc_call(body, (
        jax.ShapeDtypeStruct((n, 1), jnp.float32),
        jax.ShapeDtypeStruct((n, hdim), jnp.float32),
    ))(degp, hmat)


def _tc_layer1_combine(aggp, hp, dinv, w2, b1, n):
    def body(aggp_ref, hp_ref, dinv_ref, w2_ref, b1_ref, gp_ref):
        agg = aggp_ref[0, :n, :] + aggp_ref[1, :n, :]
        dv = dinv_ref[...]
        a1 = jnp.maximum(dv * (agg + hp_ref[...]) + b1_ref[...], 0.0)
        g = jnp.dot(a1, w2_ref[...], preferred_element_type=jnp.float32)
        gp_ref[...] = dv * g
    return _tc_call(body, jax.ShapeDtypeStruct((n, 1), jnp.float32))(
        aggp, hp, dinv, w2, b1)


def _tc_layer2_combine(agg2p, gp, dinv, b2, n):
    def body(a2_ref, gp_ref, dinv_ref, b2_ref, o_ref):
        agg2 = jnp.sum(a2_ref[...], axis=0)[:n]
        o_ref[...] = dinv_ref[...] * (agg2[:, None] + gp_ref[...]) + b2_ref[...]
    return _tc_call(body, jax.ShapeDtypeStruct((n, 1), jnp.float32))(
        agg2p, gp, dinv, b2)


def kernel(x, edge_index, edge_weight, W1, b1, W2, b2):
    n, d = x.shape
    h = W1.shape[1]
    e = edge_index.shape[1]
    np_ = ((n + 127) // 128) * 128
    if np_ % (NS * 8) != 0:
        np_ = ((n + NS * 8 - 1) // (NS * 8)) * (NS * 8)

    src = edge_index[0].astype(jnp.int32)
    dst = edge_index[1].astype(jnp.int32)
    ew = edge_weight.astype(jnp.float32)

    degp = _build_sc_scalar_agg(e, n, np_, gather=False)(dst, ew)
    hmat = _tc_matmul(x, W1)

    dinv, hp = _tc_norm(degp, hmat, n)

    aggp = _build_sc_row_agg(e, n, np_, h)(src, dst, ew, hp)

    gp = _tc_layer1_combine(aggp, hp, dinv, W2, b1.reshape(1, h), n)

    agg2p = _build_sc_scalar_agg(e, n, np_, gather=True)(
        src, dst, ew, gp.reshape(n))

    out = _tc_layer2_combine(agg2p, gp, dinv, b2.reshape(1, 1), n)
    return out[:, 0]


# speedup vs baseline: 61.7786x; 3.7553x over previous
import jax
import jax.numpy as jnp
from jax import lax
from jax.experimental import pallas as pl
from jax.experimental.pallas import tpu as pltpu
from jax.experimental.pallas import tpu_sc as plsc

NC = 2
NS = 16
L = 16
NW = NC * NS
FG = 4
ES = NW // FG

_SC_PARAMS = pltpu.CompilerParams(
    needs_layout_passes=False, use_tc_tiling_on_sc=False)


def _build_sc_scalar_agg(e, n, np_, gather):
    ew_per_worker = e // NW
    assert e % NW == 0 and ew_per_worker % L == 0
    mesh = plsc.VectorSubcoreMesh(core_axis_name="c", subcore_axis_name="s")

    scratch = [
        pltpu.VMEM((ew_per_worker,), jnp.int32),
        pltpu.VMEM((ew_per_worker,), jnp.float32),
        pltpu.VMEM((np_,), jnp.float32),
        pltpu.SemaphoreType.DMA,
    ]
    if gather:
        scratch += [
            pltpu.VMEM((ew_per_worker,), jnp.int32),
            pltpu.VMEM((n,), jnp.float32),
        ]

    def body(*refs):
        if gather:
            (src_hbm, dst_hbm, ew_hbm, tab_hbm, out_hbm,
             didx, evals, acc, sem, sidx, tab) = refs
        else:
            (dst_hbm, ew_hbm, out_hbm, didx, evals, acc, sem) = refs
        cid = lax.axis_index("c")
        sid = lax.axis_index("s")
        wid = sid * NC + cid
        base = wid * ew_per_worker

        pltpu.async_copy(dst_hbm.at[pl.ds(base, ew_per_worker)], didx, sem)
        pltpu.async_copy(ew_hbm.at[pl.ds(base, ew_per_worker)], evals, sem)
        if gather:
            pltpu.async_copy(src_hbm.at[pl.ds(base, ew_per_worker)], sidx, sem)
            pltpu.async_copy(tab_hbm, tab, sem)

        def zero(i, carry):
            acc[pl.ds(i * L, L)] = jnp.zeros((L,), jnp.float32)
            return carry
        lax.fori_loop(0, np_ // L, zero, 0)

        pltpu.make_async_copy(dst_hbm.at[pl.ds(0, ew_per_worker)], didx, sem).wait()
        pltpu.make_async_copy(ew_hbm.at[pl.ds(0, ew_per_worker)], evals, sem).wait()
        if gather:
            pltpu.make_async_copy(src_hbm.at[pl.ds(0, ew_per_worker)], sidx, sem).wait()
            pltpu.make_async_copy(tab_hbm, tab, sem).wait()

        def grp(g, carry):
            dv = didx[pl.ds(g * L, L)]
            ev = evals[pl.ds(g * L, L)]
            if gather:
                sv = sidx[pl.ds(g * L, L)]
                ev = ev * plsc.load_gather(tab, [sv])
            plsc.addupdate_scatter(acc, [dv], ev)
            return carry
        lax.fori_loop(0, ew_per_worker // L, grp, 0)

        pltpu.sync_copy(acc, out_hbm.at[wid])

    return pl.kernel(
        body,
        out_type=jax.ShapeDtypeStruct((NW, np_), jnp.float32),
        mesh=mesh,
        scratch_types=scratch,
        compiler_params=_SC_PARAMS,
    )


def _build_sc_row_agg(e, n, np_, ch):
    rows = e // ch
    assert e % ch == 0 and rows % ES == 0 and ch % L == 0
    cpt = rows // ES
    mesh = plsc.VectorSubcoreMesh(core_axis_name="c", subcore_axis_name="s")

    scratch = [
        pltpu.VMEM((FG * n,), jnp.float32),
        pltpu.VMEM((FG * np_,), jnp.float32),
        pltpu.VMEM((ch,), jnp.int32),
        pltpu.VMEM((ch,), jnp.int32),
        pltpu.VMEM((ch,), jnp.int32),
        pltpu.VMEM((ch,), jnp.int32),
        pltpu.VMEM((ch,), jnp.float32),
        pltpu.VMEM((ch,), jnp.float32),
        pltpu.SemaphoreType.DMA,
        pltpu.SemaphoreType.DMA,
    ]

    def body(src2d, dst2d, ew2d, hpt4, out_hbm,
             hq, acc, sb0, sb1, db0, db1, eb0, eb1, sem0, sem1):
        cid = lax.axis_index("c")
        sid = lax.axis_index("s")
        wid = sid * NC + cid
        fg = wid % FG
        es = wid // FG
        sb = (sb0, sb1)
        db = (db0, db1)
        eb = (eb0, eb1)
        sems = (sem0, sem1)

        def start(b, ci):
            row = es * cpt + ci
            pltpu.async_copy(src2d.at[row], sb[b], sems[b])
            pltpu.async_copy(dst2d.at[row], db[b], sems[b])
            pltpu.async_copy(ew2d.at[row], eb[b], sems[b])

        def drain(b):
            pltpu.make_async_copy(src2d.at[0], sb[b], sems[b]).wait()
            pltpu.make_async_copy(dst2d.at[0], db[b], sems[b]).wait()
            pltpu.make_async_copy(ew2d.at[0], eb[b], sems[b]).wait()

        start(0, 0)
        pltpu.sync_copy(hpt4.at[fg], hq)

        def zero(i, carry):
            acc[pl.ds(i * L, L)] = jnp.zeros((L,), jnp.float32)
            return carry
        lax.fori_loop(0, (FG * np_) // L, zero, 0)

        coff_n = [jnp.full((L,), c * n, jnp.int32) for c in range(FG)]
        coff_p = [jnp.full((L,), c * np_, jnp.int32) for c in range(FG)]

        def outer(k, carry):
            for b in range(2):
                ci = k * 2 + b
                drain(b)

                @pl.when(ci + 1 < cpt)
                def _():
                    start(1 - b, ci + 1)

                def grp(g, c2):
                    sv = sb[b][pl.ds(g * L, L)]
                    dv = db[b][pl.ds(g * L, L)]
                    ev = eb[b][pl.ds(g * L, L)]
                    for c in range(FG):
                        gv = plsc.load_gather(hq, [sv + coff_n[c]])
                        plsc.addupdate_scatter(acc, [dv + coff_p[c]], gv * ev)
                    return c2
                lax.fori_loop(0, ch // L, grp, 0)
            return carry
        lax.fori_loop(0, cpt // 2, outer, 0)

        pltpu.sync_copy(acc, out_hbm.at[wid])

    return pl.kernel(
        body,
        out_type=jax.ShapeDtypeStruct((NW, FG * np_), jnp.float32),
        mesh=mesh,
        scratch_types=scratch,
        compiler_params=_SC_PARAMS,
    )


def _tc_call(body, out_shapes):
    return pl.pallas_call(body, out_shape=out_shapes)


def _tc_matmul_t(w1, x):
    def body(w_ref, x_ref, o_ref):
        o_ref[...] = lax.dot_general(
            w_ref[...], x_ref[...], (((0,), (1,)), ((), ())),
            preferred_element_type=jnp.float32)
    n = x.shape[0]
    h = w1.shape[1]
    return _tc_call(body, jax.ShapeDtypeStruct((h, n), jnp.float32))(w1, x)


def _tc_norm(degp, hmat_t, n):
    def body(degp_ref, h_ref, dinv_ref, hp_ref):
        deg = jnp.sum(degp_ref[...], axis=0) + 1.0
        dinv = lax.rsqrt(deg)[:n][None, :]
        dinv_ref[...] = dinv
        hp_ref[...] = h_ref[...] * dinv
    hdim = hmat_t.shape[0]
    return _tc_call(body, (
        jax.ShapeDtypeStruct((1, n), jnp.float32),
        jax.ShapeDtypeStruct((hdim, n), jnp.float32),
    ))(degp, hmat_t)


def _tc_layer1_combine(aggp, hp_t, dinv_t, w2, b1c, n, np_, h):
    def body(aggp_ref, hp_ref, dinv_ref, w2_ref, b1_ref, gp_ref):
        a = aggp_ref[...].reshape(ES, FG, FG, np_)
        agg_t = jnp.sum(a, axis=0).reshape(h, np_)[:, :n]
        dv = dinv_ref[...]
        a1 = jnp.maximum(dv * (agg_t + hp_ref[...]) + b1_ref[...], 0.0)
        g = lax.dot_general(w2_ref[...], a1, (((0,), (0,)), ((), ())),
                            preferred_element_type=jnp.float32)
        gp_ref[...] = dv * g
    return _tc_call(body, jax.ShapeDtypeStruct((1, n), jnp.float32))(
        aggp, hp_t, dinv_t, w2, b1c)


def _tc_layer2_combine(agg2p, gp_t, dinv_t, b2c, n):
    def body(a2_ref, gp_ref, dinv_ref, b2_ref, o_ref):
        agg2 = jnp.sum(a2_ref[...], axis=0)[:n][None, :]
        o_ref[...] = dinv_ref[...] * (agg2 + gp_ref[...]) + b2_ref[...]
    return _tc_call(body, jax.ShapeDtypeStruct((1, n), jnp.float32))(
        agg2p, gp_t, dinv_t, b2c)


def kernel(x, edge_index, edge_weight, W1, b1, W2, b2):
    n, d = x.shape
    h = W1.shape[1]
    e = edge_index.shape[1]
    assert h == FG * FG
    np_ = ((n + 127) // 128) * 128

    src = edge_index[0].astype(jnp.int32)
    dst = edge_index[1].astype(jnp.int32)
    ew = edge_weight.astype(jnp.float32)

    ch = 4000
    while e % ch != 0 or (e // ch) % ES != 0 or ((e // ch) // ES) % 2 != 0:
        ch -= 8

    degp = _build_sc_scalar_agg(e, n, np_, gather=False)(dst, ew)
    hmat_t = _tc_matmul_t(W1, x)

    dinv_t, hp_t = _tc_norm(degp, hmat_t, n)

    aggp = _build_sc_row_agg(e, n, np_, ch)(
        src.reshape(e // ch, ch), dst.reshape(e // ch, ch),
        ew.reshape(e // ch, ch), hp_t.reshape(FG, FG * n))

    gp_t = _tc_layer1_combine(
        aggp.reshape(NW, FG, np_), hp_t, dinv_t, W2, b1.reshape(h, 1),
        n, np_, h)

    agg2p = _build_sc_scalar_agg(e, n, np_, gather=True)(
        src, dst, ew, gp_t.reshape(n))

    out = _tc_layer2_combine(agg2p, gp_t, dinv_t, b2.reshape(1, 1), n)
    return out[0]


# speedup vs baseline: 101.1834x; 1.6378x over previous
import jax
import jax.numpy as jnp
from jax import lax
from jax.experimental import pallas as pl
from jax.experimental.pallas import tpu as pltpu
from jax.experimental.pallas import tpu_sc as plsc

NC = 2
NS = 16
L = 16
NW = NC * NS
FG = 4
ES = NW // FG

_SC_PARAMS = pltpu.CompilerParams(
    needs_layout_passes=False, use_tc_tiling_on_sc=False)


def _build_sc_scalar_agg(e, n, np_, gather):
    ew_per_worker = e // NW
    assert e % NW == 0 and ew_per_worker % L == 0
    mesh = plsc.VectorSubcoreMesh(core_axis_name="c", subcore_axis_name="s")

    scratch = [
        pltpu.VMEM((ew_per_worker,), jnp.int32),
        pltpu.VMEM((ew_per_worker,), jnp.float32),
        pltpu.VMEM((np_,), jnp.float32),
        pltpu.SemaphoreType.DMA,
    ]
    if gather:
        scratch += [
            pltpu.VMEM((ew_per_worker,), jnp.int32),
            pltpu.VMEM((n,), jnp.float32),
        ]

    def body(*refs):
        if gather:
            (src_hbm, dst_hbm, ew_hbm, tab_hbm, out_hbm,
             didx, evals, acc, sem, sidx, tab) = refs
        else:
            (dst_hbm, ew_hbm, out_hbm, didx, evals, acc, sem) = refs
        cid = lax.axis_index("c")
        sid = lax.axis_index("s")
        wid = sid * NC + cid
        base = wid * ew_per_worker

        pltpu.async_copy(dst_hbm.at[pl.ds(base, ew_per_worker)], didx, sem)
        pltpu.async_copy(ew_hbm.at[pl.ds(base, ew_per_worker)], evals, sem)
        if gather:
            pltpu.async_copy(src_hbm.at[pl.ds(base, ew_per_worker)], sidx, sem)
            pltpu.async_copy(tab_hbm, tab, sem)

        @plsc.parallel_loop(0, np_ // L, unroll=8)
        def zero(i):
            acc[pl.ds(i * L, L)] = jnp.zeros((L,), jnp.float32)

        pltpu.make_async_copy(dst_hbm.at[pl.ds(0, ew_per_worker)], didx, sem).wait()
        pltpu.make_async_copy(ew_hbm.at[pl.ds(0, ew_per_worker)], evals, sem).wait()
        if gather:
            pltpu.make_async_copy(src_hbm.at[pl.ds(0, ew_per_worker)], sidx, sem).wait()
            pltpu.make_async_copy(tab_hbm, tab, sem).wait()

        @plsc.parallel_loop(0, ew_per_worker // L, unroll=4)
        def grp(g):
            dv = didx[pl.ds(g * L, L)]
            ev = evals[pl.ds(g * L, L)]
            if gather:
                sv = sidx[pl.ds(g * L, L)]
                ev = ev * plsc.load_gather(tab, [sv])
            plsc.addupdate_scatter(acc, [dv], ev)

        pltpu.sync_copy(acc, out_hbm.at[wid])

    return pl.kernel(
        body,
        out_type=jax.ShapeDtypeStruct((NW, np_), jnp.float32),
        mesh=mesh,
        scratch_types=scratch,
        compiler_params=_SC_PARAMS,
    )


def _build_sc_row_agg(e, n, np_, ch):
    rows = e // ch
    assert e % ch == 0 and rows % ES == 0 and ch % L == 0
    cpt = rows // ES
    mesh = plsc.VectorSubcoreMesh(core_axis_name="c", subcore_axis_name="s")

    scratch = [
        pltpu.VMEM((FG * n,), jnp.float32),
        pltpu.VMEM((FG * np_,), jnp.float32),
        pltpu.VMEM((ch,), jnp.int32),
        pltpu.VMEM((ch,), jnp.int32),
        pltpu.VMEM((ch,), jnp.int32),
        pltpu.VMEM((ch,), jnp.int32),
        pltpu.VMEM((ch,), jnp.float32),
        pltpu.VMEM((ch,), jnp.float32),
        pltpu.SemaphoreType.DMA,
        pltpu.SemaphoreType.DMA,
        pltpu.SemaphoreType.DMA,
    ]

    def body(src2d, dst2d, ew2d, hpt4, out_hbm,
             hq, acc, sb0, sb1, db0, db1, eb0, eb1, sem0, sem1, semh):
        cid = lax.axis_index("c")
        sid = lax.axis_index("s")
        wid = sid * NC + cid
        fg = wid % FG
        es = wid // FG
        sb = (sb0, sb1)
        db = (db0, db1)
        eb = (eb0, eb1)
        sems = (sem0, sem1)

        def start(b, ci):
            row = es * cpt + ci
            pltpu.async_copy(src2d.at[row], sb[b], sems[b])
            pltpu.async_copy(dst2d.at[row], db[b], sems[b])
            pltpu.async_copy(ew2d.at[row], eb[b], sems[b])

        def drain(b):
            pltpu.make_async_copy(src2d.at[0], sb[b], sems[b]).wait()
            pltpu.make_async_copy(dst2d.at[0], db[b], sems[b]).wait()
            pltpu.make_async_copy(ew2d.at[0], eb[b], sems[b]).wait()

        start(0, 0)
        pltpu.async_copy(hpt4.at[fg], hq, semh)

        @plsc.parallel_loop(0, (FG * np_) // L, unroll=8)
        def zero(i):
            acc[pl.ds(i * L, L)] = jnp.zeros((L,), jnp.float32)

        pltpu.make_async_copy(hpt4.at[0], hq, semh).wait()

        coff_n = [jnp.full((L,), c * n, jnp.int32) for c in range(FG)]
        coff_p = [jnp.full((L,), c * np_, jnp.int32) for c in range(FG)]

        def outer(k, carry):
            for b in range(2):
                ci = k * 2 + b
                drain(b)

                @pl.when(ci + 1 < cpt)
                def _():
                    start(1 - b, ci + 1)

                @plsc.parallel_loop(0, ch // L, unroll=4)
                def grp(g):
                    sv = sb[b][pl.ds(g * L, L)]
                    dv = db[b][pl.ds(g * L, L)]
                    ev = eb[b][pl.ds(g * L, L)]
                    for c in range(FG):
                        gv = plsc.load_gather(hq, [sv + coff_n[c]])
                        plsc.addupdate_scatter(acc, [dv + coff_p[c]], gv * ev)
            return carry
        lax.fori_loop(0, cpt // 2, outer, 0)

        pltpu.sync_copy(acc, out_hbm.at[wid])

    return pl.kernel(
        body,
        out_type=jax.ShapeDtypeStruct((NW, FG * np_), jnp.float32),
        mesh=mesh,
        scratch_types=scratch,
        compiler_params=_SC_PARAMS,
    )


def _tc_call(body, out_shapes):
    return pl.pallas_call(body, out_shape=out_shapes)


def _tc_matmul_t(w1, x):
    def body(w_ref, x_ref, o_ref):
        o_ref[...] = lax.dot_general(
            w_ref[...], x_ref[...], (((0,), (1,)), ((), ())),
            preferred_element_type=jnp.float32)
    n = x.shape[0]
    h = w1.shape[1]
    return _tc_call(body, jax.ShapeDtypeStruct((h, n), jnp.float32))(w1, x)


def _tc_norm(degp, hmat_t, n):
    def body(degp_ref, h_ref, dinv_ref, hp_ref):
        deg = jnp.sum(degp_ref[...], axis=0) + 1.0
        dinv = lax.rsqrt(deg)[:n][None, :]
        dinv_ref[...] = dinv
        hp_ref[...] = h_ref[...] * dinv
    hdim = hmat_t.shape[0]
    return _tc_call(body, (
        jax.ShapeDtypeStruct((1, n), jnp.float32),
        jax.ShapeDtypeStruct((hdim, n), jnp.float32),
    ))(degp, hmat_t)


def _tc_layer1_combine(aggp, hp_t, dinv_t, w2, b1c, n, np_, h):
    def body(aggp_ref, hp_ref, dinv_ref, w2_ref, b1_ref, gp_ref):
        a = aggp_ref[...].reshape(ES, FG, FG, np_)
        agg_t = jnp.sum(a, axis=0).reshape(h, np_)[:, :n]
        dv = dinv_ref[...]
        a1 = jnp.maximum(dv * (agg_t + hp_ref[...]) + b1_ref[...], 0.0)
        g = lax.dot_general(w2_ref[...], a1, (((0,), (0,)), ((), ())),
                            preferred_element_type=jnp.float32)
        gp_ref[...] = dv * g
    return _tc_call(body, jax.ShapeDtypeStruct((1, n), jnp.float32))(
        aggp, hp_t, dinv_t, w2, b1c)


def _tc_layer2_combine(agg2p, gp_t, dinv_t, b2c, n):
    def body(a2_ref, gp_ref, dinv_ref, b2_ref, o_ref):
        agg2 = jnp.sum(a2_ref[...], axis=0)[:n][None, :]
        o_ref[...] = dinv_ref[...] * (agg2 + gp_ref[...]) + b2_ref[...]
    return _tc_call(body, jax.ShapeDtypeStruct((1, n), jnp.float32))(
        agg2p, gp_t, dinv_t, b2c)


def kernel(x, edge_index, edge_weight, W1, b1, W2, b2):
    n, d = x.shape
    h = W1.shape[1]
    e = edge_index.shape[1]
    assert h == FG * FG
    np_ = ((n + 127) // 128) * 128

    src = edge_index[0].astype(jnp.int32)
    dst = edge_index[1].astype(jnp.int32)
    ew = edge_weight.astype(jnp.float32)

    ch = 4000
    while e % ch != 0 or (e // ch) % ES != 0 or ((e // ch) // ES) % 2 != 0:
        ch -= 8

    degp = _build_sc_scalar_agg(e, n, np_, gather=False)(dst, ew)
    hmat_t = _tc_matmul_t(W1, x)

    dinv_t, hp_t = _tc_norm(degp, hmat_t, n)

    aggp = _build_sc_row_agg(e, n, np_, ch)(
        src.reshape(e // ch, ch), dst.reshape(e // ch, ch),
        ew.reshape(e // ch, ch), hp_t.reshape(FG, FG * n))

    gp_t = _tc_layer1_combine(
        aggp.reshape(NW, FG, np_), hp_t, dinv_t, W2, b1.reshape(h, 1),
        n, np_, h)

    agg2p = _build_sc_scalar_agg(e, n, np_, gather=True)(
        src, dst, ew, gp_t.reshape(n))

    out = _tc_layer2_combine(agg2p, gp_t, dinv_t, b2.reshape(1, 1), n)
    return out[0]
